# baseline (speedup 1.0000x reference)
import functools

import jax
import jax.numpy as jnp
from jax import lax
from jax.experimental import pallas as pl
from jax.experimental.pallas import tpu as pltpu
from jax.experimental.pallas import tpu_sc as plsc

_NUM_CORES = 2
_NUM_SUBCORES = 16
_NUM_WORKERS = _NUM_CORES * _NUM_SUBCORES
_CHUNK = 128
_PREP_BLK = 8192
_OUT_BLK = 512


def kernel(indices, weight):
    batch, seq = indices.shape
    vocab, dim = weight.shape
    n = batch * seq
    per_worker = n // _NUM_WORKERS
    n_chunks = per_worker // _CHUNK

    flat_idx = indices.T.reshape(1, n).astype(jnp.int32)

    eye = jnp.eye(dim, dtype=weight.dtype)

    def _prep_body(i_ref, e_ref, o_ref):
        x = i_ref[...]
        o_ref[:, :dim] = jax.lax.dot_general(
            x,
            e_ref[...],
            (((0,), (0,)), ((), ())),
            preferred_element_type=jnp.float32,
        )

    w_pad = pl.pallas_call(
        _prep_body,
        grid=((vocab + _PREP_BLK - 1) // _PREP_BLK,),
        in_specs=[
            pl.BlockSpec((dim, _PREP_BLK), lambda i: (0, i)),
            pl.BlockSpec((dim, dim), lambda i: (0, 0)),
        ],
        out_specs=pl.BlockSpec((_PREP_BLK, 128), lambda i: (i, 0)),
        out_shape=jax.ShapeDtypeStruct((vocab, 128), weight.dtype),
    )(weight.T, eye)

    mesh = plsc.VectorSubcoreMesh(core_axis_name="c", subcore_axis_name="s")

    @functools.partial(
        pl.kernel,
        out_type=jax.ShapeDtypeStruct((n, 128), weight.dtype),
        mesh=mesh,
        scratch_types=[
            pltpu.VMEM((per_worker,), jnp.int32),
            pltpu.VMEM((2, _CHUNK, 128), jnp.float32),
            pltpu.SemaphoreType.DMA,
            pltpu.SemaphoreType.DMA,
            pltpu.SemaphoreType.DMA,
            pltpu.SemaphoreType.DMA,
        ],
    )
    def gather_kernel(w_hbm, i_hbm, o_hbm, idx_v, g_v, gs0, gs1, ws0, ws1):
        gsem = (gs0, gs1)
        wsem = (ws0, ws1)

        wid = lax.axis_index("s") * _NUM_CORES + lax.axis_index("c")
        base = wid * per_worker
        pltpu.sync_copy(i_hbm.at[0, pl.ds(base, per_worker)], idx_v)

        def start_gather(slot, c):
            pltpu.async_copy(
                w_hbm.at[idx_v.at[pl.ds(c * _CHUNK, _CHUNK)]],
                g_v.at[slot],
                gsem[slot],
            )

        def wait_gather(slot, c):
            pltpu.make_async_copy(
                w_hbm.at[idx_v.at[pl.ds(c * _CHUNK, _CHUNK)]],
                g_v.at[slot],
                gsem[slot],
            ).wait()

        def start_write(slot, c):
            pltpu.async_copy(
                g_v.at[slot],
                o_hbm.at[pl.ds(base + c * _CHUNK, _CHUNK)],
                wsem[slot],
            )

        def wait_write(slot, c):
            pltpu.make_async_copy(
                g_v.at[slot],
                o_hbm.at[pl.ds(base + c * _CHUNK, _CHUNK)],
                wsem[slot],
            ).wait()

        start_gather(0, 0)
        start_gather(1, 1)

        @pl.loop(0, n_chunks, step=2)
        def _(c):
            for b in range(2):
                cc = c + b
                wait_gather(b, cc)
                start_write(b, cc)

                @pl.when(cc + 2 < n_chunks)
                def _():
                    wait_write(b, cc)
                    start_gather(b, cc + 2)

        wait_write(0, n_chunks - 2)
        wait_write(1, n_chunks - 1)

    rows = gather_kernel(w_pad, flat_idx)

    eye_b = jnp.eye(_OUT_BLK, dtype=weight.dtype)

    def _out_body(i_ref, e_ref, o_ref):
        x = i_ref[...]
        y = jax.lax.dot_general(
            x[:, :dim],
            e_ref[...],
            (((0,), (0,)), ((), ())),
            preferred_element_type=jnp.float32,
        )
        o_ref[...] = y.reshape(1, dim, _OUT_BLK)

    out3 = pl.pallas_call(
        _out_body,
        grid=(seq, batch // _OUT_BLK),
        in_specs=[
            pl.BlockSpec(
                (_OUT_BLK, 128),
                lambda s_, j: (s_ * (batch // _OUT_BLK) + j, 0),
            ),
            pl.BlockSpec((_OUT_BLK, _OUT_BLK), lambda s_, j: (0, 0)),
        ],
        out_specs=pl.BlockSpec((1, dim, _OUT_BLK), lambda s_, j: (s_, 0, j)),
        out_shape=jax.ShapeDtypeStruct((seq, dim, batch), weight.dtype),
    )(rows, eye_b)

    return out3.transpose(2, 0, 1)


# speedup vs baseline: 1.2659x; 1.2659x over previous
import functools

import jax
import jax.numpy as jnp
from jax import lax
from jax.experimental import pallas as pl
from jax.experimental.pallas import tpu as pltpu
from jax.experimental.pallas import tpu_sc as plsc

_NUM_CORES = 2
_NUM_SUBCORES = 16
_NUM_WORKERS = _NUM_CORES * _NUM_SUBCORES
_CHUNK = 128
_PREP_BLK = 8192


def kernel(indices, weight):
    batch, seq = indices.shape
    vocab, dim = weight.shape
    n = batch * seq
    per_worker = n // _NUM_WORKERS
    n_chunks = per_worker // _CHUNK

    flat_idx = indices.reshape(1, n).astype(jnp.int32)

    eye = jnp.eye(dim, dtype=weight.dtype)

    def _prep_body(i_ref, e_ref, o_ref):
        x = i_ref[...]
        o_ref[:, :dim] = jax.lax.dot_general(
            x,
            e_ref[...],
            (((0,), (0,)), ((), ())),
            preferred_element_type=jnp.float32,
        )

    w_pad = pl.pallas_call(
        _prep_body,
        grid=((vocab + _PREP_BLK - 1) // _PREP_BLK,),
        in_specs=[
            pl.BlockSpec((dim, _PREP_BLK), lambda i: (0, i)),
            pl.BlockSpec((dim, dim), lambda i: (0, 0)),
        ],
        out_specs=pl.BlockSpec((_PREP_BLK, 128), lambda i: (i, 0)),
        out_shape=jax.ShapeDtypeStruct((vocab, 128), weight.dtype),
    )(weight.T, eye)

    mesh = plsc.VectorSubcoreMesh(core_axis_name="c", subcore_axis_name="s")

    @functools.partial(
        pl.kernel,
        out_type=jax.ShapeDtypeStruct((n, 128), weight.dtype),
        mesh=mesh,
        scratch_types=[
            pltpu.VMEM((per_worker,), jnp.int32),
            pltpu.VMEM((2, _CHUNK, 128), jnp.float32),
            pltpu.SemaphoreType.DMA,
            pltpu.SemaphoreType.DMA,
            pltpu.SemaphoreType.DMA,
            pltpu.SemaphoreType.DMA,
        ],
    )
    def gather_kernel(w_hbm, i_hbm, o_hbm, idx_v, g_v, gs0, gs1, ws0, ws1):
        gsem = (gs0, gs1)
        wsem = (ws0, ws1)

        wid = lax.axis_index("s") * _NUM_CORES + lax.axis_index("c")
        base = wid * per_worker
        pltpu.sync_copy(i_hbm.at[0, pl.ds(base, per_worker)], idx_v)

        def start_gather(slot, c):
            pltpu.async_copy(
                w_hbm.at[idx_v.at[pl.ds(c * _CHUNK, _CHUNK)]],
                g_v.at[slot],
                gsem[slot],
            )

        def wait_gather(slot, c):
            pltpu.make_async_copy(
                w_hbm.at[idx_v.at[pl.ds(c * _CHUNK, _CHUNK)]],
                g_v.at[slot],
                gsem[slot],
            ).wait()

        def start_write(slot, c):
            pltpu.async_copy(
                g_v.at[slot],
                o_hbm.at[pl.ds(base + c * _CHUNK, _CHUNK)],
                wsem[slot],
            )

        def wait_write(slot, c):
            pltpu.make_async_copy(
                g_v.at[slot],
                o_hbm.at[pl.ds(base + c * _CHUNK, _CHUNK)],
                wsem[slot],
            ).wait()

        start_gather(0, 0)
        start_gather(1, 1)

        @pl.loop(0, n_chunks, step=2)
        def _(c):
            for b in range(2):
                cc = c + b
                wait_gather(b, cc)
                start_write(b, cc)

                @pl.when(cc + 2 < n_chunks)
                def _():
                    wait_write(b, cc)
                    start_gather(b, cc + 2)

        wait_write(0, n_chunks - 2)
        wait_write(1, n_chunks - 1)

    rows = gather_kernel(w_pad, flat_idx)
    return rows[:, :dim].reshape(batch, seq, dim)
